# Initial kernel scaffold; baseline (speedup 1.0000x reference)
import jax
import jax.numpy as jnp
from jax.experimental import pallas as pl


def kernel(x):
    raise NotImplementedError("write your pallas kernel here")


# speedup vs baseline: 3.1488x; 3.1488x over previous
import jax
import jax.numpy as jnp
from jax.experimental import pallas as pl
from jax.experimental.pallas import tpu as pltpu


def _l2norm(x, eps=1e-12):
    d = jnp.sqrt(jnp.sum(x * x, axis=(2, 3), keepdims=True))
    return x / jnp.maximum(d, eps)


def _stats(x):
    rgb, ir = x[0], x[1]
    rgb_cap = jnp.mean(rgb, axis=1, keepdims=True)
    rgb_cmp = jnp.max(rgb, axis=1, keepdims=True)
    ir_cap = jnp.mean(ir, axis=1, keepdims=True)
    ir_cmp = jnp.max(ir, axis=1, keepdims=True)
    x1_cp = jnp.concatenate([rgb_cap, rgb_cmp], axis=1)
    x2_cp = jnp.concatenate([ir_cap, ir_cmp], axis=1)
    cp = x1_cp + x2_cp
    sa = jnp.maximum(cp[:, ::2, :, :], cp[:, 1::2, :, :])
    sa_sig = jax.nn.sigmoid(sa)
    sa_norm = _l2norm(sa_sig)
    sim_rgb = jnp.sum(sa_norm * _l2norm(rgb), axis=(2, 3))
    sim_ir = jnp.sum(sa_norm * _l2norm(ir), axis=(2, 3))
    return sa, sim_rgb, sim_ir


def _ksort(srow_ref, scol_ref, sims_ref, pos_ref, pmeta_ref, idxm_ref):
    C = srow_ref.shape[3]
    s = pl.program_id(0)
    srow = srow_ref[0, 0]
    scol = scol_ref[0, 0]
    iota_row = jax.lax.broadcasted_iota(jnp.int32, (1, C), 1)
    iota_col = jax.lax.broadcasted_iota(jnp.int32, (C, 1), 0)
    lt = scol < srow
    eq = scol == srow
    before = lt | (eq & (iota_col < iota_row))
    rank = jnp.sum(jnp.where(before, 1.0, 0.0), axis=0,
                   keepdims=True).astype(jnp.int32)

    allsims = sims_ref[...]
    cnt = jnp.sum(jnp.where(allsims > 0, 1.0, 0.0), axis=3)
    k0 = jnp.max(cnt[0]).astype(jnp.int32)
    k1 = jnp.max(cnt[1]).astype(jnp.int32)
    is0 = s == 0
    act = jnp.where(is0, (k1 > k0) & (k0 > 0), (k0 > k1) & (k1 > 0))
    kk = jnp.where(is0, k0, k1)

    pos_act = jnp.where(rank < kk, rank,
                        jnp.where(rank == C - 1, kk, rank + 1))
    pos = jnp.where(act, pos_act, rank)
    pos_ref[0, 0] = pos

    ppos = jnp.where(act, kk, 0)
    acti = act.astype(jnp.int32)
    lanes = jax.lax.broadcasted_iota(jnp.int32, (1, 128), 1)
    pmeta_ref[0, 0] = jnp.where(lanes == 0, ppos,
                                jnp.where(lanes == 1, acti, 0))
    idxm = jnp.sum(jnp.where(rank == 0, iota_row, 0))
    idxm_ref[0, 0] = jnp.zeros((1, 128), jnp.int32) + idxm


def _kperm(idx_ref, x_ref, rowa_ref, rowb_ref, pos_ref, pmeta_ref, sig_ref,
           out_ref):
    C = x_ref.shape[2]
    s = pl.program_id(0)
    xb = x_ref[0, 0]
    posr = pos_ref[0, 0]
    iota_col = jax.lax.broadcasted_iota(jnp.int32, (C, 1), 0)
    P = (posr == iota_col).astype(jnp.float32)
    out = jax.lax.dot_general(
        P, xb, (((1,), (0,)), ((), ())),
        preferred_element_type=jnp.float32)
    ra = rowa_ref[0, 0, 0]
    rb = rowb_ref[0, 0, 0]
    own = jnp.where(s == 0, ra, rb)
    act = pmeta_ref[0, 0, 0, 1] != 0
    prow = jnp.where(act, jnp.maximum(ra, rb), own)
    pp = pmeta_ref[0, 0, 0, 0]
    out = jnp.where(iota_col == pp, prow, out)
    out_ref[0, 0] = out * sig_ref[0, 0]


def kernel(x):
    S, B, C, H, W = x.shape
    HW = H * W
    f32 = jnp.float32

    sa, sim_rgb, sim_ir = _stats(x)
    sa_sig = jax.nn.sigmoid(sa)
    sims = jnp.stack([sim_rgb, sim_ir]).reshape(S, B, 1, C)
    sims_col = sims.reshape(S, B, C, 1)
    sig_arr = sa_sig.reshape(B, 1, HW)
    xr = x.reshape(S, B, C, HW)
    xr5 = x.reshape(S, B, C, 1, HW)

    pos, pmeta, idxm = pl.pallas_call(
        _ksort,
        grid=(S, B),
        in_specs=[
            pl.BlockSpec((1, 1, 1, C), lambda s, b: (s, b, 0, 0)),
            pl.BlockSpec((1, 1, C, 1), lambda s, b: (s, b, 0, 0)),
            pl.BlockSpec((S, B, 1, C), lambda s, b: (0, 0, 0, 0)),
        ],
        out_specs=[pl.BlockSpec((1, 1, 1, C), lambda s, b: (s, b, 0, 0)),
                   pl.BlockSpec((1, 1, 1, 128), lambda s, b: (s, b, 0, 0)),
                   pl.BlockSpec((1, 1, 1, 128), lambda s, b: (s, b, 0, 0))],
        out_shape=[jax.ShapeDtypeStruct((S, B, 1, C), jnp.int32),
                   jax.ShapeDtypeStruct((S, B, 1, 128), jnp.int32),
                   jax.ShapeDtypeStruct((S, B, 1, 128), jnp.int32)],
    )(sims, sims_col, sims)

    grid_spec = pltpu.PrefetchScalarGridSpec(
        num_scalar_prefetch=1,
        grid=(S, B),
        in_specs=[
            pl.BlockSpec((1, 1, C, HW), lambda s, b, idx: (s, b, 0, 0)),
            pl.BlockSpec((1, 1, 1, 1, HW),
                         lambda s, b, idx: (0, b, idx[0, b, 0, 0], 0, 0)),
            pl.BlockSpec((1, 1, 1, 1, HW),
                         lambda s, b, idx: (1, b, idx[1, b, 0, 0], 0, 0)),
            pl.BlockSpec((1, 1, 1, C), lambda s, b, idx: (s, b, 0, 0)),
            pl.BlockSpec((1, 1, 1, 128), lambda s, b, idx: (s, b, 0, 0)),
            pl.BlockSpec((1, 1, HW), lambda s, b, idx: (b, 0, 0)),
        ],
        out_specs=pl.BlockSpec((1, 1, C, HW), lambda s, b, idx: (s, b, 0, 0)),
    )
    out = pl.pallas_call(
        _kperm,
        grid_spec=grid_spec,
        out_shape=jax.ShapeDtypeStruct((S, B, C, HW), f32),
    )(idxm, xr, xr5, xr5, pos, pmeta, sig_arr)

    out = out.reshape(S, B, C, H, W)
    return out[0], out[1]
